# baseline (speedup 1.0000x reference)
import functools

import jax
import jax.numpy as jnp
from jax.experimental import pallas as pl
from jax.experimental.pallas import tpu as pltpu

_G = 128
_BN = 4096


def _leaky(x):
    return jnp.where(x >= 0, x, 0.01 * x)


def _valid_rows(i, bn, n):
    row = i * bn + jax.lax.broadcasted_iota(jnp.int32, (bn, 1), 0)
    return row < n


def _gate_pool_kernel(h_ref, bcol_ref, W1_ref, b1_ref, W2_ref, b2_ref,
                      W3_ref, b3_ref, g_ref, m_out_ref, s_out_ref,
                      poolT_ref, m_ref, s_ref, pT_ref, *, nb, G, n):
    i = pl.program_id(0)

    @pl.when(i == 0)
    def _():
        m_ref[...] = jnp.full_like(m_ref, -1e30)
        s_ref[...] = jnp.zeros_like(s_ref)
        pT_ref[...] = jnp.zeros_like(pT_ref)

    bn = h_ref.shape[0]
    hb = jnp.where(_valid_rows(i, bn, n), h_ref[...], 0.0)
    x = _leaky(jnp.dot(hb, W1_ref[...],
                       preferred_element_type=jnp.float32) + b1_ref[...])
    x = _leaky(jnp.dot(x, W2_ref[...],
                       preferred_element_type=jnp.float32) + b2_ref[...])
    g = jnp.dot(x, W3_ref[...],
                preferred_element_type=jnp.float32) + b3_ref[...]
    g_ref[...] = g

    ids = bcol_ref[...]
    seg = jax.lax.broadcasted_iota(jnp.int32, (1, G), 1)
    hit = ids == seg
    maskf = hit.astype(jnp.float32)

    m_old = m_ref[...]
    m_blk = jnp.max(jnp.where(hit, g, -1e30), axis=0, keepdims=True)
    m_new = jnp.maximum(m_old, m_blk)
    scale = jnp.exp(m_old - m_new)
    m_gather = jnp.sum(maskf * m_new, axis=1, keepdims=True)
    e = jnp.exp(g - m_gather)
    s_blk = jnp.sum(maskf * e, axis=0, keepdims=True)
    s_ref[...] = s_ref[...] * scale + s_blk
    m_ref[...] = m_new

    hs = hb * e
    part = jax.lax.dot_general(hs, maskf, (((0,), (0,)), ((), ())),
                               preferred_element_type=jnp.float32)
    pT_ref[...] = pT_ref[...] * scale + part

    @pl.when(i == nb - 1)
    def _():
        m_out_ref[...] = m_ref[...]
        s_fin = s_ref[...]
        s_out_ref[...] = s_fin
        poolT_ref[...] = pT_ref[...] / jnp.where(s_fin > 0, s_fin, 1.0)


def _scores_kernel(bcol_ref, g_ref, m_ref, s_ref, scores_ref, *, G):
    ids = bcol_ref[...]
    seg = jax.lax.broadcasted_iota(jnp.int32, (1, G), 1)
    maskf = (ids == seg).astype(jnp.float32)
    m = m_ref[...]
    s = s_ref[...]
    m_gather = jnp.sum(maskf * m, axis=1, keepdims=True)
    s_gather = jnp.sum(maskf * s, axis=1, keepdims=True)
    e = jnp.exp(g_ref[...] - m_gather)
    scores_ref[...] = e / jnp.where(s_gather > 0, s_gather, 1.0)


def kernel(h, batch, W1, b1, W2, b2, W3, b3):
    N, D = h.shape
    H = W1.shape[1]
    G = _G
    BN = _BN
    nb = -(-N // BN)
    npad = nb * BN

    bp = jnp.pad(batch, (0, npad - N), constant_values=G)
    bcol = bp.reshape(npad, 1)
    b1r = b1.reshape(1, H)
    b2r = b2.reshape(1, H)
    b3r = b3.reshape(1, 1)

    g, m, s, poolT = pl.pallas_call(
        functools.partial(_gate_pool_kernel, nb=nb, G=G, n=N),
        grid=(nb,),
        in_specs=[
            pl.BlockSpec((BN, D), lambda i: (i, 0)),
            pl.BlockSpec((BN, 1), lambda i: (i, 0)),
            pl.BlockSpec((D, H), lambda i: (0, 0)),
            pl.BlockSpec((1, H), lambda i: (0, 0)),
            pl.BlockSpec((H, H), lambda i: (0, 0)),
            pl.BlockSpec((1, H), lambda i: (0, 0)),
            pl.BlockSpec((H, 1), lambda i: (0, 0)),
            pl.BlockSpec((1, 1), lambda i: (0, 0)),
        ],
        out_specs=[
            pl.BlockSpec((BN, 1), lambda i: (i, 0)),
            pl.BlockSpec((1, G), lambda i: (0, 0)),
            pl.BlockSpec((1, G), lambda i: (0, 0)),
            pl.BlockSpec((D, G), lambda i: (0, 0)),
        ],
        out_shape=[
            jax.ShapeDtypeStruct((npad, 1), jnp.float32),
            jax.ShapeDtypeStruct((1, G), jnp.float32),
            jax.ShapeDtypeStruct((1, G), jnp.float32),
            jax.ShapeDtypeStruct((D, G), jnp.float32),
        ],
        scratch_shapes=[
            pltpu.VMEM((1, G), jnp.float32),
            pltpu.VMEM((1, G), jnp.float32),
            pltpu.VMEM((D, G), jnp.float32),
        ],
    )(h, bcol, W1, b1r, W2, b2r, W3, b3r)

    scores = pl.pallas_call(
        functools.partial(_scores_kernel, G=G),
        grid=(nb,),
        in_specs=[
            pl.BlockSpec((BN, 1), lambda i: (i, 0)),
            pl.BlockSpec((BN, 1), lambda i: (i, 0)),
            pl.BlockSpec((1, G), lambda i: (0, 0)),
            pl.BlockSpec((1, G), lambda i: (0, 0)),
        ],
        out_specs=pl.BlockSpec((BN, 1), lambda i: (i, 0)),
        out_shape=jax.ShapeDtypeStruct((N, 1), jnp.float32),
    )(bcol, g, m, s)

    return (poolT.T, scores)


# speedup vs baseline: 1.6262x; 1.6262x over previous
import functools

import jax
import jax.numpy as jnp
from jax.experimental import pallas as pl
from jax.experimental.pallas import tpu as pltpu

_G = 128
_BN = 4096


def _leaky(x):
    return jnp.where(x >= 0, x, 0.01 * x)


def _fused_kernel(h_ref, brow_ref, W1_ref, b1_ref, W2_ref, b2_ref,
                  W3_ref, b3_ref, scores_ref, pool_ref,
                  m_ref, s_ref, p_ref, g_ref, *, nb, G, n):
    i = pl.program_id(0)
    bn = h_ref.shape[0]

    @pl.when(i == 0)
    def _():
        m_ref[...] = jnp.full_like(m_ref, -1e30)
        s_ref[...] = jnp.zeros_like(s_ref)
        p_ref[...] = jnp.zeros_like(p_ref)

    ids_row = brow_ref[0]
    seg_col = jax.lax.broadcasted_iota(jnp.int32, (G, 1), 0)
    hit = seg_col == ids_row
    maskf = hit.astype(jnp.float32)

    @pl.when(i < nb)
    def _phase0():
        row = i * bn + jax.lax.broadcasted_iota(jnp.int32, (bn, 1), 0)
        hb = jnp.where(row < n, h_ref[...], 0.0)
        x = _leaky(jnp.dot(hb, W1_ref[...],
                           preferred_element_type=jnp.float32) + b1_ref[...])
        x = _leaky(jnp.dot(x, W2_ref[...],
                           preferred_element_type=jnp.float32) + b2_ref[...])
        g = jax.lax.dot_general(W3_ref[...], x, (((0,), (1,)), ((), ())),
                                preferred_element_type=jnp.float32)
        g = g + b3_ref[...]
        g_ref[pl.ds(i, 1), :] = g

        m_old = m_ref[...]
        m_blk = jnp.max(jnp.where(hit, g, -1e30), axis=1, keepdims=True)
        m_new = jnp.maximum(m_old, m_blk)
        scale = jnp.exp(m_old - m_new)
        m_gath = jnp.sum(maskf * m_new, axis=0, keepdims=True)
        e = jnp.exp(g - m_gath)
        w = maskf * e
        s_blk = jnp.sum(w, axis=1, keepdims=True)
        s_ref[...] = s_ref[...] * scale + s_blk
        m_ref[...] = m_new
        part = jnp.dot(w, hb, preferred_element_type=jnp.float32)
        p_ref[...] = p_ref[...] * scale + part

        @pl.when(i == nb - 1)
        def _():
            s_fin = s_ref[...]
            pool_ref[...] = p_ref[...] / jnp.where(s_fin > 0, s_fin, 1.0)

    @pl.when(i >= nb)
    def _phase1():
        j = i - nb
        g = g_ref[pl.ds(j, 1), :]
        s = s_ref[...]
        c = m_ref[...] + jnp.log(jnp.where(s > 0, s, 1.0))
        c_gath = jnp.sum(maskf * c, axis=0, keepdims=True)
        sc = jnp.exp(g - c_gath)
        scores_ref[...] = jnp.transpose(sc)


def kernel(h, batch, W1, b1, W2, b2, W3, b3):
    N, D = h.shape
    H = W1.shape[1]
    G = _G
    BN = _BN
    nb = -(-N // BN)
    npad = nb * BN

    brow = jnp.pad(batch, (0, npad - N), constant_values=G).reshape(nb, 1, BN)
    b1r = b1.reshape(1, H)
    b2r = b2.reshape(1, H)
    b3r = b3.reshape(1, 1)

    scores, pool = pl.pallas_call(
        functools.partial(_fused_kernel, nb=nb, G=G, n=N),
        grid=(2 * nb,),
        in_specs=[
            pl.BlockSpec((BN, D), lambda i: (jnp.where(i < nb, i, 0), 0)),
            pl.BlockSpec((1, 1, BN),
                         lambda i: (jnp.where(i < nb, i, i - nb), 0, 0)),
            pl.BlockSpec((D, H), lambda i: (0, 0)),
            pl.BlockSpec((1, H), lambda i: (0, 0)),
            pl.BlockSpec((H, H), lambda i: (0, 0)),
            pl.BlockSpec((1, H), lambda i: (0, 0)),
            pl.BlockSpec((H, 1), lambda i: (0, 0)),
            pl.BlockSpec((1, 1), lambda i: (0, 0)),
        ],
        out_specs=[
            pl.BlockSpec((BN, 1), lambda i: (jnp.where(i < nb, 0, i - nb), 0)),
            pl.BlockSpec((G, D), lambda i: (0, 0)),
        ],
        out_shape=[
            jax.ShapeDtypeStruct((N, 1), jnp.float32),
            jax.ShapeDtypeStruct((G, D), jnp.float32),
        ],
        scratch_shapes=[
            pltpu.VMEM((G, 1), jnp.float32),
            pltpu.VMEM((G, 1), jnp.float32),
            pltpu.VMEM((G, D), jnp.float32),
            pltpu.VMEM((nb, BN), jnp.float32),
        ],
    )(h, brow, W1, b1r, W2, b2r, W3, b3r)

    return (pool, scores)
